# baseline (speedup 1.0000x reference)
import functools

import jax
import jax.numpy as jnp
from jax import lax
from jax.experimental import pallas as pl
from jax.experimental.pallas import tpu as pltpu
from jax.experimental.pallas import tpu_sc as plsc

_CLS = 512
_N = 50000
_EPS = 1e-07
_BLK = 5000

_NC, _NS, _L = 2, 16, 16
_NW = _NC * _NS
_CHUNKS = 98
_PERW = _CHUNKS * _L
_NPAD = _NW * _PERW
_SENT = 2 * _CLS
_ROWS = 80


def _argmax_body(yp_ref, yt_ref, out_ref):
    v = yp_ref[...]
    m = jnp.max(v, axis=1, keepdims=True)
    iota = lax.broadcasted_iota(jnp.int32, v.shape, 1)
    masked = jnp.where(v == m, iota, _CLS)
    pred = jnp.min(masked, axis=1, keepdims=True)
    correct = (pred == yt_ref[...]).astype(jnp.int32)
    out_ref[...] = pred + _CLS * correct


def _sc_hist(idx_hbm, out_hbm, idx_v, bins_v, merged_v):
    wid = lax.axis_index("s") * _NC + lax.axis_index("c")
    pltpu.sync_copy(idx_hbm.at[wid], idx_v)

    zero16 = jnp.zeros((_L,), jnp.float32)
    nbins = _ROWS * _L

    def zbody(j, _):
        for r in range(_L):
            bins_v[pl.ds(r * nbins + j * _L, _L)] = zero16
        return 0

    lax.fori_loop(0, _ROWS, zbody, 0)

    ones = jnp.ones((_L,), jnp.float32)
    lanebase = lax.iota(jnp.int32, _L) * nbins

    def sbody(c, _):
        idx = idx_v[c, :]
        plsc.addupdate_scatter(bins_v, [lanebase + idx], ones)
        return 0

    lax.fori_loop(0, _CHUNKS, sbody, 0)

    def mbody(j, _):
        acc = bins_v[pl.ds(j * _L, _L)]
        for r in range(1, _L):
            acc = acc + bins_v[pl.ds(r * nbins + j * _L, _L)]
        merged_v[j, :] = acc
        return 0

    lax.fori_loop(0, _ROWS, mbody, 0)
    pltpu.sync_copy(merged_v, out_hbm.at[wid])


_sc_hist_call = functools.partial(
    pl.kernel,
    out_type=jax.ShapeDtypeStruct((_NW, _ROWS, _L), jnp.float32),
    mesh=plsc.VectorSubcoreMesh(core_axis_name="c", subcore_axis_name="s"),
    compiler_params=pltpu.CompilerParams(needs_layout_passes=False),
    scratch_types=[
        pltpu.VMEM((_CHUNKS, _L), jnp.int32),
        pltpu.VMEM((_L * _ROWS * _L,), jnp.float32),
        pltpu.VMEM((_ROWS, _L), jnp.float32),
    ],
)(_sc_hist)


def _fin_body(h_ref, out_ref):
    h = jnp.sum(h_ref[...], axis=0)
    a = h[0:32, :]
    b = h[32:64, :]
    ratio = b / (a + b + _EPS)
    s = jnp.sum(ratio, axis=1, keepdims=True)
    out_ref[...] = jnp.sum(s, axis=0, keepdims=True) / _CLS


def kernel(y_pred, y_true):
    yt = y_true.astype(jnp.int32).reshape(_N, 1)
    comb = pl.pallas_call(
        _argmax_body,
        grid=(_N // _BLK,),
        in_specs=[
            pl.BlockSpec((_BLK, _CLS), lambda i: (i, 0)),
            pl.BlockSpec((_BLK, 1), lambda i: (i, 0)),
        ],
        out_specs=pl.BlockSpec((_BLK, 1), lambda i: (i, 0)),
        out_shape=jax.ShapeDtypeStruct((_N, 1), jnp.int32),
    )(y_pred, yt)
    pad = jnp.full((_NPAD - _N,), _SENT, jnp.int32)
    idx = jnp.concatenate([comb.reshape(_N), pad]).reshape(_NW, _CHUNKS, _L)
    parts = _sc_hist_call(idx)
    out = pl.pallas_call(
        _fin_body,
        out_shape=jax.ShapeDtypeStruct((1, 1), jnp.float32),
    )(parts)
    return out[0, 0]


# speedup vs baseline: 1.3823x; 1.3823x over previous
import functools

import jax
import jax.numpy as jnp
from jax import lax
from jax.experimental import pallas as pl
from jax.experimental.pallas import tpu as pltpu
from jax.experimental.pallas import tpu_sc as plsc

_CLS = 512
_N = 50000
_EPS = 1e-07

_NC, _NS, _L = 2, 16, 16
_NW = _NC * _NS
_GRP = 20
_RPW = _GRP * _L
_NSC = _NW * _RPW
_NT = _N - _NSC
_BLK = 3976
_ROWS = 80
_NBINS = _ROWS * _L
_CHUNKS = _CLS // _L


def _tc_body(yp_ref, yt_ref, out_ref, cnt_ref, cor_ref):
    i = pl.program_id(0)

    @pl.when(i == 0)
    def _init():
        cnt_ref[...] = jnp.zeros_like(cnt_ref)
        cor_ref[...] = jnp.zeros_like(cor_ref)

    v = yp_ref[...]
    m = jnp.max(v, axis=1, keepdims=True)
    eqf = (v == m).astype(jnp.float32)
    iota = lax.broadcasted_iota(jnp.int32, v.shape, 1)
    teqf = (iota == yt_ref[...]).astype(jnp.float32)
    cnt_ref[...] += jnp.sum(eqf, axis=0, keepdims=True)
    cor_ref[...] += jnp.sum(eqf * teqf, axis=0, keepdims=True)

    @pl.when(i == pl.num_programs(0) - 1)
    def _fin():
        out_ref[0:1, :] = cnt_ref[...]
        out_ref[1:2, :] = cor_ref[...]


def _sc_body(yp_hbm, yt_hbm, out_hbm, rows_v, yt_v, bins_v, merged_v):
    wid = lax.axis_index("s") * _NC + lax.axis_index("c")
    base = _NT + wid * _RPW

    zero16 = jnp.zeros((_L,), jnp.float32)

    def zbody(j, _):
        for r in range(_L):
            bins_v[pl.ds(r * _NBINS + j * _L, _L)] = zero16
        return 0

    lax.fori_loop(0, _ROWS, zbody, 0)

    ones = jnp.ones((_L,), jnp.float32)
    lanes = lax.iota(jnp.int32, _L)
    lanebase = lanes * _NBINS

    def gbody(g, _):
        row0 = base + g * _L
        pltpu.sync_copy(yp_hbm.at[pl.ds(row0, _L), :], rows_v)
        pltpu.sync_copy(yt_hbm.at[pl.ds(row0, _L)], yt_v)
        pvec = jnp.zeros((_L,), jnp.int32)
        for r in range(_L):
            acc_m = rows_v[r, pl.ds(0, _L)]
            acc_c = jnp.zeros((_L,), jnp.int32)
            for c in range(1, _CHUNKS):
                vc = rows_v[r, pl.ds(c * _L, _L)]
                take = vc > acc_m
                acc_m = jnp.where(take, vc, acc_m)
                acc_c = jnp.where(take, c, acc_c)
            mx = jnp.max(acc_m)
            idxs = acc_c * _L + lanes
            cand = jnp.where(acc_m == mx, idxs, _CLS)
            pred = jnp.min(cand)
            pvec = jnp.where(lanes == r, pred, pvec)
        tvec = yt_v[...]
        bin_ = pvec + jnp.where(pvec == tvec, _CLS, 0)
        plsc.addupdate_scatter(bins_v, [lanebase + bin_], ones)
        return 0

    lax.fori_loop(0, _GRP, gbody, 0)

    def mbody(j, _):
        acc = bins_v[pl.ds(j * _L, _L)]
        for r in range(1, _L):
            acc = acc + bins_v[pl.ds(r * _NBINS + j * _L, _L)]
        merged_v[pl.ds(j * _L, _L)] = acc
        return 0

    lax.fori_loop(0, _ROWS, mbody, 0)
    pltpu.sync_copy(merged_v, out_hbm.at[wid])


_sc_call = functools.partial(
    pl.kernel,
    out_type=jax.ShapeDtypeStruct((_NW, _NBINS), jnp.float32),
    mesh=plsc.VectorSubcoreMesh(core_axis_name="c", subcore_axis_name="s"),
    compiler_params=pltpu.CompilerParams(needs_layout_passes=False),
    scratch_types=[
        pltpu.VMEM((_L, _CLS), jnp.float32),
        pltpu.VMEM((_L,), jnp.int32),
        pltpu.VMEM((_L * _NBINS,), jnp.float32),
        pltpu.VMEM((_NBINS,), jnp.float32),
    ],
)(_sc_body)


def _fin_body(tc_ref, sc_ref, out_ref):
    h = jnp.sum(sc_ref[...], axis=0, keepdims=True)
    b = h[:, _CLS:2 * _CLS]
    cnt = h[:, 0:_CLS] + b + tc_ref[0:1, :]
    cor = b + tc_ref[1:2, :]
    ratio = cor / (cnt + _EPS)
    out_ref[...] = jnp.sum(ratio, axis=1, keepdims=True) / _CLS


def kernel(y_pred, y_true):
    yt = y_true.astype(jnp.int32)
    sc_parts = _sc_call(y_pred, yt)
    tc_parts = pl.pallas_call(
        _tc_body,
        grid=(_NT // _BLK,),
        in_specs=[
            pl.BlockSpec((_BLK, _CLS), lambda i: (i, 0)),
            pl.BlockSpec((_BLK, 1), lambda i: (i, 0)),
        ],
        out_specs=pl.BlockSpec((2, _CLS), lambda i: (0, 0)),
        out_shape=jax.ShapeDtypeStruct((2, _CLS), jnp.float32),
        scratch_shapes=[
            pltpu.VMEM((1, _CLS), jnp.float32),
            pltpu.VMEM((1, _CLS), jnp.float32),
        ],
    )(y_pred, yt.reshape(_N, 1))
    out = pl.pallas_call(
        _fin_body,
        out_shape=jax.ShapeDtypeStruct((1, 1), jnp.float32),
    )(tc_parts, sc_parts)
    return out[0, 0]
